# Initial kernel scaffold; baseline (speedup 1.0000x reference)
import jax
import jax.numpy as jnp
from jax.experimental import pallas as pl


def kernel(pointcloud, params):
    raise NotImplementedError("write your pallas kernel here")


# speedup vs baseline: 11.6328x; 11.6328x over previous
import functools
from typing import Sequence

import jax
import jax.numpy as jnp
from jax.experimental import pallas as pl
from jax.experimental.pallas import tpu as pltpu

_DEFAULT = jax.lax.Precision.DEFAULT
_HIGHEST = jax.lax.Precision.HIGHEST


def _dot(a, b, precision):
    return jax.lax.dot_general(a, b, (((1,), (1,)), ((), ())),
                               precision=precision,
                               preferred_element_type=jnp.float32)


def _matmul(a, b, precision):
    return jax.lax.dot_general(a, b, (((1,), (0,)), ((), ())),
                               precision=precision,
                               preferred_element_type=jnp.float32)


def _fps_kernel(xs_ref, ys_ref, zs_ref, ox_ref, oy_ref, oz_ref, *, npoint):
    B, N = xs_ref.shape
    xs, ys, zs = xs_ref[...], ys_ref[...], zs_ref[...]
    lin = jax.lax.broadcasted_iota(jnp.int32, (B, N), 1)
    lin_o = jax.lax.broadcasted_iota(jnp.int32, (B, npoint), 1)

    def body(t, carry):
        dist, far, ox, oy, oz = carry
        eq = lin == far
        cx = jnp.sum(jnp.where(eq, xs, 0.0), axis=-1, keepdims=True)
        cy = jnp.sum(jnp.where(eq, ys, 0.0), axis=-1, keepdims=True)
        cz = jnp.sum(jnp.where(eq, zs, 0.0), axis=-1, keepdims=True)
        sel = lin_o == t
        ox = jnp.where(sel, cx, ox)
        oy = jnp.where(sel, cy, oy)
        oz = jnp.where(sel, cz, oz)
        d = (xs - cx) ** 2 + (ys - cy) ** 2 + (zs - cz) ** 2
        dist = jnp.minimum(dist, d)
        m = jnp.max(dist, axis=-1, keepdims=True)
        far = jnp.min(jnp.where(dist == m, lin, N), axis=-1, keepdims=True)
        return dist, far, ox, oy, oz

    init = (jnp.full((B, N), 1e10, jnp.float32),
            jnp.zeros((B, 1), jnp.int32),
            jnp.zeros((B, npoint), jnp.float32),
            jnp.zeros((B, npoint), jnp.float32),
            jnp.zeros((B, npoint), jnp.float32))
    _, _, ox, oy, oz = jax.lax.fori_loop(0, npoint, body, init)
    ox_ref[...] = ox
    oy_ref[...] = oy
    oz_ref[...] = oz


def _fps_call(xyz, npoint):
    B, N, _ = xyz.shape
    xs = xyz[:, :, 0]
    ys = xyz[:, :, 1]
    zs = xyz[:, :, 2]
    out = jax.ShapeDtypeStruct((B, npoint), jnp.float32)
    ox, oy, oz = pl.pallas_call(
        functools.partial(_fps_kernel, npoint=npoint),
        out_shape=(out, out, out),
    )(xs, ys, zs)
    return jnp.stack([ox, oy, oz], axis=-1)


def _sa_kernel(new_ref, xyz_ref, xyzT_ref, datG_hi_ref, datG_lo_ref, *wb_refs,
               o_ref, G_scratch, lo_scratch, radius, nsample, nlayers):
    St = new_ref.shape[1]
    N = xyz_ref.shape[1]
    C8 = datG_hi_ref.shape[2]
    C = C8 // 8
    new_t = new_ref[0]
    xyz = xyz_ref[0]

    d = -2.0 * _dot(new_t, xyz, _DEFAULT)
    d = d + jnp.sum(new_t ** 2, axis=-1, keepdims=True)
    d = d + jnp.sum(xyzT_ref[0] ** 2, axis=0, keepdims=True)

    iota_n = jax.lax.broadcasted_iota(jnp.int32, (St, N), 1)
    w = jnp.where(d <= radius * radius, iota_n, N)
    iota_c = jax.lax.broadcasted_iota(jnp.int32, (St, N // 8), 1)

    idx0 = jnp.min(w, axis=-1, keepdims=True)
    wcur = w
    for k in range(nsample):
        if k == 0:
            idx_k = idx0
        else:
            mk = jnp.min(wcur, axis=-1, keepdims=True)
            idx_k = jnp.where(mk == N, idx0, mk)
        wcur = jnp.where(wcur == idx_k, N, wcur)
        chunk = idx_k // 8
        G_scratch[k * St:(k + 1) * St, :] = (chunk == iota_c).astype(jnp.float32)
        lo_scratch[k * St:(k + 1) * St, :] = idx_k - 8 * chunk

    G = G_scratch[...]
    lo3 = lo_scratch[...]
    coarse = _matmul(G, datG_hi_ref[0], _DEFAULT) + \
        _matmul(G, datG_lo_ref[0], _DEFAULT)
    rows = jnp.zeros((nsample * St, C), jnp.float32)
    for r in range(8):
        rows = jnp.where(lo3 == r, coarse[:, r * C:(r + 1) * C], rows)

    ctr = jnp.broadcast_to(new_t[None], (nsample, St, 3)).reshape(nsample * St, 3)
    h = jnp.concatenate([rows[:, :3] - ctr, rows[:, 3:]], axis=-1)
    for li in range(nlayers):
        W = wb_refs[2 * li][...]
        b = wb_refs[2 * li + 1][...]
        h = jnp.maximum(_matmul(h, W, _DEFAULT) + b, 0.0)

    Cout = h.shape[-1]
    acc = h[0:St]
    for k in range(1, nsample):
        acc = jnp.maximum(acc, h[k * St:(k + 1) * St])
    o_ref[0] = acc


def _sa_call(new_xyz, xyz, feats, radius, nsample, layers, St):
    B, S, _ = new_xyz.shape
    N = xyz.shape[1]
    data = xyz if feats is None else jnp.concatenate([xyz, feats], axis=-1)
    C = data.shape[-1]
    hi = (data.astype(jnp.bfloat16)).astype(jnp.float32)
    lo = data - hi
    datG_hi = hi.reshape(B, N // 8, 8 * C)
    datG_lo = lo.reshape(B, N // 8, 8 * C)
    Cout = layers[-1][0].shape[1]
    wb = []
    in_specs = [
        pl.BlockSpec((1, St, 3), lambda b, s: (b, s, 0)),
        pl.BlockSpec((1, N, 3), lambda b, s: (b, 0, 0)),
        pl.BlockSpec((1, 3, N), lambda b, s: (b, 0, 0)),
        pl.BlockSpec((1, N // 8, 8 * C), lambda b, s: (b, 0, 0)),
        pl.BlockSpec((1, N // 8, 8 * C), lambda b, s: (b, 0, 0)),
    ]
    for (W, bb) in layers:
        wb.append(W)
        wb.append(bb.reshape(1, -1))
        in_specs.append(pl.BlockSpec(W.shape, lambda b, s: (0, 0)))
        in_specs.append(pl.BlockSpec((1, bb.shape[0]), lambda b, s: (0, 0)))
    kfn = functools.partial(
        _sa_kernel, radius=radius, nsample=nsample, nlayers=len(layers))

    def body(*refs):
        kfn(*refs[:-3], o_ref=refs[-3], G_scratch=refs[-2], lo_scratch=refs[-1])

    return pl.pallas_call(
        body,
        grid=(B, S // St),
        in_specs=in_specs,
        out_specs=pl.BlockSpec((1, St, Cout), lambda b, s: (b, s, 0)),
        out_shape=jax.ShapeDtypeStruct((B, S, Cout), jnp.float32),
        scratch_shapes=[pltpu.VMEM((nsample * St, N // 8), jnp.float32),
                        pltpu.VMEM((nsample * St, 1), jnp.int32)],
    )(new_xyz, xyz, jnp.swapaxes(xyz, 1, 2), datG_hi, datG_lo, *wb)


def _fp_kernel(unk_ref, known_ref, knownT_ref, kfeat_ref, ufeat_ref, *wb_refs,
               o_ref, nlayers, nhead):
    St = unk_ref.shape[1]
    M = known_ref.shape[1]
    unk = unk_ref[0]
    known = known_ref[0]

    d = -2.0 * _dot(unk, known, _DEFAULT)
    d = d + jnp.sum(unk ** 2, axis=-1, keepdims=True)
    d = d + jnp.sum(knownT_ref[0] ** 2, axis=0, keepdims=True)

    iota_m = jax.lax.broadcasted_iota(jnp.int32, (St, M), 1)
    recs = []
    Es = []
    for _ in range(3):
        mv = jnp.min(d, axis=-1, keepdims=True)
        idx = jnp.min(jnp.where(d == mv, iota_m, M), axis=-1, keepdims=True)
        E = idx == iota_m
        recs.append(1.0 / (jnp.maximum(mv, 0.0) + 1e-8))
        Es.append(E)
        d = jnp.where(E, 1e30, d)
    norm = recs[0] + recs[1] + recs[2]
    A = jnp.zeros((St, M), jnp.float32)
    for E, rc in zip(Es, recs):
        A = A + jnp.where(E, rc / norm, 0.0)
    interp = _matmul(A, kfeat_ref[0], _HIGHEST)

    h = jnp.concatenate([interp, ufeat_ref[0]], axis=-1)
    for li in range(nlayers):
        W = wb_refs[2 * li][...]
        b = wb_refs[2 * li + 1][...]
        h = jnp.maximum(_matmul(h, W, _DEFAULT) + b, 0.0)
    if nhead:
        W = wb_refs[2 * nlayers][...]
        b = wb_refs[2 * nlayers + 1][...]
        h = jnp.maximum(_matmul(h, W, _DEFAULT) + b, 0.0)
        W = wb_refs[2 * nlayers + 2][...]
        b = wb_refs[2 * nlayers + 3][...]
        h = _matmul(h, W, _DEFAULT) + b
    o_ref[0] = h


def _fp_call(unknown, known, unknown_feats, known_feats, layers, St,
             head=None):
    B, S, _ = unknown.shape
    M = known.shape[1]
    Cf = known_feats.shape[-1]
    Cu = unknown_feats.shape[-1]
    wb = []
    in_specs = [
        pl.BlockSpec((1, St, 3), lambda b, s: (b, s, 0)),
        pl.BlockSpec((1, M, 3), lambda b, s: (b, 0, 0)),
        pl.BlockSpec((1, 3, M), lambda b, s: (b, 0, 0)),
        pl.BlockSpec((1, M, Cf), lambda b, s: (b, 0, 0)),
        pl.BlockSpec((1, St, Cu), lambda b, s: (b, s, 0)),
    ]
    all_layers = list(layers) + (list(head) if head else [])
    for (W, bb) in all_layers:
        wb.append(W)
        wb.append(bb.reshape(1, -1))
        in_specs.append(pl.BlockSpec(W.shape, lambda b, s: (0, 0)))
        in_specs.append(pl.BlockSpec((1, bb.shape[0]), lambda b, s: (0, 0)))
    Cout = all_layers[-1][0].shape[1]
    kfn = functools.partial(_fp_kernel, nlayers=len(layers),
                            nhead=len(head) if head else 0)

    def body(*refs):
        kfn(*refs[:-1], o_ref=refs[-1])

    return pl.pallas_call(
        body,
        grid=(B, S // St),
        in_specs=in_specs,
        out_specs=pl.BlockSpec((1, St, Cout), lambda b, s: (b, s, 0)),
        out_shape=jax.ShapeDtypeStruct((B, S, Cout), jnp.float32),
    )(unknown, known, jnp.swapaxes(known, 1, 2), known_feats,
      unknown_feats, *wb)


def kernel(pointcloud, params):
    xyz = pointcloud[..., 0:3]
    feats = pointcloud[..., 3:]

    l1_xyz = _fps_call(xyz, 1024)
    l1_f = _sa_call(l1_xyz, xyz, feats, 0.1, 32, params['sa1'], St=128)
    l2_xyz = _fps_call(l1_xyz, 256)
    l2_f = _sa_call(l2_xyz, l1_xyz, l1_f, 0.2, 32, params['sa2'], St=128)
    l3_xyz = _fps_call(l2_xyz, 64)
    l3_f = _sa_call(l3_xyz, l2_xyz, l2_f, 0.4, 32, params['sa3'], St=64)
    l4_xyz = _fps_call(l3_xyz, 16)
    l4_f = _sa_call(l4_xyz, l3_xyz, l3_f, 0.8, 32, params['sa4'], St=16)

    l3_f = _fp_call(l3_xyz, l4_xyz, l3_f, l4_f, params['fp4'], St=64)
    l2_f = _fp_call(l2_xyz, l3_xyz, l2_f, l3_f, params['fp3'], St=128)
    l1_f = _fp_call(l1_xyz, l2_xyz, l1_f, l2_f, params['fp2'], St=128)
    out = _fp_call(xyz, l1_xyz, feats, l1_f, params['fp1'], St=128,
                   head=params['head'])
    return jnp.transpose(out, (0, 2, 1))
